# Initial kernel scaffold; baseline (speedup 1.0000x reference)
import jax
import jax.numpy as jnp
from jax.experimental import pallas as pl


def kernel(x, pdm_exp, pdm_spa, lexp, lspa, weight_exp, weight_spa, bias_spa_en, bias_exp_en, bias_spa_de, bias_exp_de):
    raise NotImplementedError("write your pallas kernel here")


# speedup vs baseline: 1.2352x; 1.2352x over previous
import jax
import jax.numpy as jnp
from jax.experimental import pallas as pl
from jax.experimental.pallas import tpu as pltpu

_BM = 512


def _elu2(v):
    return jnp.where(v > 0.0, v, 2.0 * (jnp.exp(v) - 1.0))


def _adj(pdm_ref, scale_ref):
    p = pdm_ref[...]
    return jnp.exp(scale_ref[0, 0] * p * p)


def _mm_body(x_ref, w_ref, out_ref):
    out_ref[...] = jnp.dot(x_ref[...], w_ref[...],
                           preferred_element_type=jnp.float32)


def _enc_body(scale_ref, pdm_ref, s_ref, b_ref, w2_ref, out_ref):
    a = _adj(pdm_ref, scale_ref)
    h = _elu2(jnp.dot(a, s_ref[...], preferred_element_type=jnp.float32)
              + b_ref[...])
    out_ref[...] = jnp.dot(h, w2_ref[...], preferred_element_type=jnp.float32)


def _enc2_body(scale_ref, pdm_ref, s_ref, b_ref, w2_ref, y_ref, s3_ref):
    a = _adj(pdm_ref, scale_ref)
    h = _elu2(jnp.dot(a, s_ref[...], preferred_element_type=jnp.float32)
              + b_ref[...])
    y_ref[...] = h
    s3_ref[...] = jnp.dot(h, w2_ref[...], preferred_element_type=jnp.float32)


def _dec_body(scale_ref, pdm_ref, s_ref, sblk_ref, b_ref, w2_ref, out_ref):
    a = _adj(pdm_ref, scale_ref)
    h = _elu2(3.0 * sblk_ref[...]
              - jnp.dot(a, s_ref[...], preferred_element_type=jnp.float32)
              + b_ref[...])
    out_ref[...] = jnp.dot(h, w2_ref[...], preferred_element_type=jnp.float32)


def _dec2_body(scale_ref, pdm_ref, s_ref, sblk_ref, b_ref, x_ref,
               pred_ref, loss_ref):
    i = pl.program_id(0)
    a = _adj(pdm_ref, scale_ref)
    p = _elu2(3.0 * sblk_ref[...]
              - jnp.dot(a, s_ref[...], preferred_element_type=jnp.float32)
              + b_ref[...])
    pred_ref[...] = p
    d = p - x_ref[...]

    @pl.when(i == 0)
    def _():
        loss_ref[...] = jnp.zeros((1, 1), jnp.float32)

    loss_ref[...] += jnp.sum(d * d).reshape(1, 1)


def kernel(x, pdm_exp, pdm_spa, lexp, lspa, weight_exp, weight_spa,
           bias_spa_en, bias_exp_en, bias_spa_de, bias_exp_de):
    n, nfeat = x.shape
    nhid = weight_exp.shape[1]
    nemb = weight_spa.shape[1]
    mblocks = n // _BM

    scale_exp = (-0.5 / (lexp * lexp)).reshape(1, 1)
    scale_spa = (-0.5 / (lspa * lspa)).reshape(1, 1)
    b_exp_en = bias_exp_en.reshape(1, nhid)
    b_spa_en = bias_spa_en.reshape(1, nemb)
    b_spa_de = bias_spa_de.reshape(1, nhid)
    b_exp_de = bias_exp_de.reshape(1, nfeat)
    w_spa_t = weight_spa.T
    w_exp_t = weight_exp.T

    s1 = pl.pallas_call(
        _mm_body,
        out_shape=jax.ShapeDtypeStruct((n, nhid), jnp.float32),
    )(x, weight_exp)

    smem_scalar = pl.BlockSpec(memory_space=pltpu.SMEM)

    def row_spec(w):
        return pl.BlockSpec((_BM, w), lambda i: (i, 0))

    def full_spec(h, w):
        return pl.BlockSpec((h, w), lambda i: (0, 0))

    s2 = pl.pallas_call(
        _enc_body,
        grid=(mblocks,),
        in_specs=[smem_scalar, row_spec(n), full_spec(n, nhid),
                  full_spec(1, nhid), full_spec(nhid, nemb)],
        out_specs=row_spec(nemb),
        out_shape=jax.ShapeDtypeStruct((n, nemb), jnp.float32),
        compiler_params=pltpu.CompilerParams(
            dimension_semantics=("arbitrary",)),
    )(scale_exp, pdm_exp, s1, b_exp_en, weight_spa)

    y, s3 = pl.pallas_call(
        _enc2_body,
        grid=(mblocks,),
        in_specs=[smem_scalar, row_spec(n), full_spec(n, nemb),
                  full_spec(1, nemb), full_spec(nemb, nhid)],
        out_specs=[row_spec(nemb), row_spec(nhid)],
        out_shape=[jax.ShapeDtypeStruct((n, nemb), jnp.float32),
                   jax.ShapeDtypeStruct((n, nhid), jnp.float32)],
        compiler_params=pltpu.CompilerParams(
            dimension_semantics=("arbitrary",)),
    )(scale_spa, pdm_spa, s2, b_spa_en, w_spa_t)

    s4 = pl.pallas_call(
        _dec_body,
        grid=(mblocks,),
        in_specs=[smem_scalar, row_spec(n), full_spec(n, nhid),
                  row_spec(nhid), full_spec(1, nhid), full_spec(nhid, nfeat)],
        out_specs=row_spec(nfeat),
        out_shape=jax.ShapeDtypeStruct((n, nfeat), jnp.float32),
        compiler_params=pltpu.CompilerParams(
            dimension_semantics=("arbitrary",)),
    )(scale_spa, pdm_spa, s3, s3, b_spa_de, w_exp_t)

    pred, loss_sum = pl.pallas_call(
        _dec2_body,
        grid=(mblocks,),
        in_specs=[smem_scalar, row_spec(n), full_spec(n, nfeat),
                  row_spec(nfeat), full_spec(1, nfeat), row_spec(nfeat)],
        out_specs=[row_spec(nfeat), pl.BlockSpec((1, 1), lambda i: (0, 0))],
        out_shape=[jax.ShapeDtypeStruct((n, nfeat), jnp.float32),
                   jax.ShapeDtypeStruct((1, 1), jnp.float32)],
        compiler_params=pltpu.CompilerParams(
            dimension_semantics=("arbitrary",)),
    )(scale_exp, pdm_exp, s4, s4, b_exp_de, x)

    loss = loss_sum[0, 0] / jnp.float32(n * nfeat)
    return (y, pred, loss)
